# Initial kernel scaffold; baseline (speedup 1.0000x reference)
import jax
import jax.numpy as jnp
from jax.experimental import pallas as pl


def kernel(x, edge_index, edge_weight, W1, b1, W2, b2):
    raise NotImplementedError("write your pallas kernel here")


# speedup vs baseline: 3.7296x; 3.7296x over previous
import functools

import jax
import jax.numpy as jnp
from jax import lax
from jax.experimental import pallas as pl
from jax.experimental.pallas import tpu as pltpu
from jax.experimental.pallas import tpu_sc as plsc

N = 10000
E = 320000
D_IN = 128
D_H = 128
D_OUT = 64

NC = 2
NS = 16
NW = NC * NS
EPT = E // NW
K = 80
CHUNKS = EPT // K
NPAD = 10240
ROWS_PT = NPAD // NS


def _make_spmm(D):
    mesh = plsc.VectorSubcoreMesh(core_axis_name="c", subcore_axis_name="s")

    @functools.partial(
        pl.kernel,
        out_type=jax.ShapeDtypeStruct((NC, NPAD, D), jnp.float32),
        mesh=mesh,
        scratch_types=[
            pltpu.VMEM((K,), jnp.int32),
            pltpu.VMEM((K,), jnp.int32),
            pltpu.VMEM((K,), jnp.float32),
            pltpu.VMEM((K, D), jnp.float32),
            pltpu.VMEM_SHARED((NPAD, D), jnp.float32),
            pltpu.SemaphoreType.DMA,
        ],
        compiler_params=pltpu.CompilerParams(use_tc_tiling_on_sc=False),
    )
    def spmm(xw_hbm, col_hbm, row_hbm, w_hbm, out_hbm, colv, rowv, wv, rows,
             acc, sem):
        c = lax.axis_index("c")
        s = lax.axis_index("s")

        def zero_body(i, _):
            for dd in range(D // 16):
                rows[i, pl.ds(dd * 16, 16)] = jnp.zeros((16,), jnp.float32)
            return 0

        lax.fori_loop(0, K, zero_body, 0)
        for j in range(ROWS_PT // K):
            pltpu.sync_copy(rows, acc.at[pl.ds(s * ROWS_PT + j * K, K)])
        plsc.subcore_barrier()

        base = (c * NS + s) * EPT

        def chunk_body(j, _):
            off = base + j * K
            pltpu.sync_copy(col_hbm.at[pl.ds(off, K)], colv)
            pltpu.sync_copy(row_hbm.at[pl.ds(off, K)], rowv)
            pltpu.sync_copy(w_hbm.at[pl.ds(off, K)], wv)
            pltpu.async_copy(xw_hbm.at[colv], rows, sem).wait()

            def mul_body(g, _):
                w16 = wv[pl.ds(g * 16, 16)]
                dnums = lax.GatherDimensionNumbers(
                    offset_dims=(), collapsed_slice_dims=(0,),
                    start_index_map=(0,))
                for l in range(16):
                    wb = lax.gather(
                        w16, jnp.full((16, 1), l, jnp.int32), dnums,
                        slice_sizes=(1,),
                        mode=lax.GatherScatterMode.PROMISE_IN_BOUNDS)
                    i = g * 16 + l
                    for dd in range(D // 16):
                        rows[i, pl.ds(dd * 16, 16)] = (
                            rows[i, pl.ds(dd * 16, 16)] * wb)
                return 0

            lax.fori_loop(0, K // 16, mul_body, 0)
            pltpu.sync_copy(rows, acc.at[rowv], add=True)
            return 0

        lax.fori_loop(0, CHUNKS, chunk_body, 0)
        plsc.subcore_barrier()

        for j in range(ROWS_PT // K):
            r0 = s * ROWS_PT + j * K
            pltpu.sync_copy(acc.at[pl.ds(r0, K)], rows)
            pltpu.sync_copy(rows, out_hbm.at[c, pl.ds(r0, K)])

    return spmm


_spmm_h = _make_spmm(D_H)
_spmm_o = _make_spmm(D_OUT)


_GRID = 10
_BN = N // _GRID


def _mm_body(x_ref, w_ref, o_ref):
    o_ref[...] = jnp.dot(x_ref[...], w_ref[...],
                         preferred_element_type=jnp.float32)


def _mid_body(p0_ref, p1_ref, b_ref, w_ref, o_ref):
    h = jnp.maximum(p0_ref[...] + p1_ref[...] + b_ref[...], 0.0)
    o_ref[...] = jnp.dot(h, w_ref[...], preferred_element_type=jnp.float32)


def _out_body(p0_ref, p1_ref, b_ref, o_ref):
    logits = p0_ref[...] + p1_ref[...] + b_ref[...]
    m = jnp.max(logits, axis=1, keepdims=True)
    lse = jnp.log(jnp.sum(jnp.exp(logits - m), axis=1, keepdims=True))
    o_ref[...] = (logits - m) - lse


def _mm(x, w):
    din, dout = w.shape
    return pl.pallas_call(
        _mm_body,
        grid=(_GRID,),
        in_specs=[
            pl.BlockSpec((_BN, din), lambda i: (i, 0)),
            pl.BlockSpec((din, dout), lambda i: (0, 0)),
        ],
        out_specs=pl.BlockSpec((_BN, dout), lambda i: (i, 0)),
        out_shape=jax.ShapeDtypeStruct((N, dout), jnp.float32),
    )(x, w)


def _mid(p0, p1, b, w):
    din, dout = w.shape
    return pl.pallas_call(
        _mid_body,
        grid=(_GRID,),
        in_specs=[
            pl.BlockSpec((_BN, din), lambda i: (i, 0)),
            pl.BlockSpec((_BN, din), lambda i: (i, 0)),
            pl.BlockSpec((1, din), lambda i: (0, 0)),
            pl.BlockSpec((din, dout), lambda i: (0, 0)),
        ],
        out_specs=pl.BlockSpec((_BN, dout), lambda i: (i, 0)),
        out_shape=jax.ShapeDtypeStruct((N, dout), jnp.float32),
    )(p0, p1, b.reshape(1, din), w)


def _fin(p0, p1, b):
    d = p0.shape[1]
    return pl.pallas_call(
        _out_body,
        grid=(_GRID,),
        in_specs=[
            pl.BlockSpec((_BN, d), lambda i: (i, 0)),
            pl.BlockSpec((_BN, d), lambda i: (i, 0)),
            pl.BlockSpec((1, d), lambda i: (0, 0)),
        ],
        out_specs=pl.BlockSpec((_BN, d), lambda i: (i, 0)),
        out_shape=jax.ShapeDtypeStruct((N, d), jnp.float32),
    )(p0, p1, b.reshape(1, d))


def kernel(x, edge_index, edge_weight, W1, b1, W2, b2):
    row = edge_index[0]
    col = edge_index[1]

    xw = _mm(x, W1)
    p1 = _spmm_h(xw, col, row, edge_weight)
    hw = _mid(p1[0, :N], p1[1, :N], b1, W2)
    p2 = _spmm_o(hw, col, row, edge_weight)
    return _fin(p2[0, :N], p2[1, :N], b2)


# speedup vs baseline: 6.1645x; 1.6529x over previous
import functools

import jax
import jax.numpy as jnp
from jax import lax
from jax.experimental import pallas as pl
from jax.experimental.pallas import tpu as pltpu
from jax.experimental.pallas import tpu_sc as plsc

N = 10000
E = 320000
D_IN = 128
D_H = 128
D_OUT = 64

NC = 2
NS = 16
EPT = E // NS
K = 80
CHUNKS = EPT // K
NBUF = 5
NPAD = 10240
ROWS_PT = NPAD // NS

_DNUMS = lax.GatherDimensionNumbers(
    offset_dims=(), collapsed_slice_dims=(0,), start_index_map=(0,))


def _bcast_lane(v16, l):
    return lax.gather(v16, jnp.full((16, 1), l, jnp.int32), _DNUMS,
                      slice_sizes=(1,),
                      mode=lax.GatherScatterMode.PROMISE_IN_BOUNDS)


def _make_spmm(DH):
    mesh = plsc.VectorSubcoreMesh(core_axis_name="c", subcore_axis_name="s")

    @functools.partial(
        pl.kernel,
        out_type=jax.ShapeDtypeStruct((NC, NPAD, DH), jnp.float32),
        mesh=mesh,
        scratch_types=[
            pltpu.VMEM((EPT,), jnp.int32),
            pltpu.VMEM((EPT,), jnp.int32),
            pltpu.VMEM((EPT,), jnp.float32),
            pltpu.VMEM((NBUF, K, DH), jnp.float32),
            pltpu.VMEM_SHARED((NPAD, DH), jnp.float32),
            pltpu.SemaphoreType.DMA((NBUF,)),
        ] + [pltpu.VMEM((K,), jnp.int32) for _ in range(2 * NBUF)],
        compiler_params=pltpu.CompilerParams(use_tc_tiling_on_sc=False),
    )
    def spmm(xw_hbm, col_hbm, row_hbm, w_hbm, out_hbm, colv, rowv, wv, gbuf,
             acc, sem, *idxbufs):
        cidx = idxbufs[:NBUF]
        ridx = idxbufs[NBUF:]
        c = lax.axis_index("c")
        s = lax.axis_index("s")
        src = xw_hbm.at[c]

        pltpu.sync_copy(col_hbm.at[s], colv)
        pltpu.sync_copy(row_hbm.at[s], rowv)
        pltpu.sync_copy(w_hbm.at[s], wv)

        def zero_body(i, _):
            for dd in range(DH // 16):
                gbuf[0, i, pl.ds(dd * 16, 16)] = jnp.zeros((16,), jnp.float32)
            return 0

        lax.fori_loop(0, K, zero_body, 0)
        for jz in range(ROWS_PT // K):
            pltpu.sync_copy(gbuf.at[0], acc.at[pl.ds(s * ROWS_PT + jz * K, K)])
        plsc.subcore_barrier()

        def _vcopy(src_ref, j_row, dst_ref):
            for q in range(K // 16):
                dst_ref[pl.ds(q * 16, 16)] = (
                    src_ref[pl.ds(j_row * K + q * 16, 16)])

        for b in range(NBUF):
            _vcopy(colv, b, cidx[b])
            pltpu.async_copy(src.at[cidx[b]], gbuf.at[b], sem.at[b])

        def outer_body(jo, _):
            for b in range(NBUF):
                j = jo * NBUF + b
                pltpu.make_async_copy(
                    src.at[cidx[b]], gbuf.at[b], sem.at[b]).wait()

                def mul_grp(g, _):
                    w16 = wv[pl.ds(j * K + g * 16, 16)]
                    for l in range(16):
                        wb = _bcast_lane(w16, l)
                        i = g * 16 + l
                        for dd in range(DH // 16):
                            gbuf[b, i, pl.ds(dd * 16, 16)] = (
                                gbuf[b, i, pl.ds(dd * 16, 16)] * wb)
                    return 0

                lax.fori_loop(0, K // 16, mul_grp, 0)
                _vcopy(rowv, j, ridx[b])
                pltpu.sync_copy(gbuf.at[b], acc.at[ridx[b]], add=True)

                @pl.when(jo < CHUNKS // NBUF - 1)
                def _():
                    _vcopy(colv, j + NBUF, cidx[b])
                    pltpu.async_copy(src.at[cidx[b]], gbuf.at[b], sem.at[b])
            return 0

        lax.fori_loop(0, CHUNKS // NBUF, outer_body, 0)
        plsc.subcore_barrier()

        for jz in range(ROWS_PT // K):
            r0 = s * ROWS_PT + jz * K
            pltpu.sync_copy(acc.at[pl.ds(r0, K)], gbuf.at[0])
            pltpu.sync_copy(gbuf.at[0], out_hbm.at[c, pl.ds(r0, K)])

    return spmm


_spmm_h = _make_spmm(D_H // 2)
_spmm_o = _make_spmm(D_OUT // 2)


_GRID = 10
_BN = N // _GRID


def _mm_body(x_ref, w_ref, o_ref):
    o_ref[0] = jnp.dot(x_ref[...], w_ref[0],
                       preferred_element_type=jnp.float32)


def _mm(x, w):
    _, din, dh = w.shape
    return pl.pallas_call(
        _mm_body,
        grid=(_GRID, 2),
        in_specs=[
            pl.BlockSpec((_BN, din), lambda i, j: (i, 0)),
            pl.BlockSpec((1, din, dh), lambda i, j: (j, 0, 0)),
        ],
        out_specs=pl.BlockSpec((1, _BN, dh), lambda i, j: (j, i, 0)),
        out_shape=jax.ShapeDtypeStruct((2, N, dh), jnp.float32),
    )(x, w)


def _mid_body(p0_ref, p1_ref, b_ref, w_ref, o_ref):
    dh = p0_ref.shape[1]
    h0 = jnp.maximum(p0_ref[...] + b_ref[0, 0, :dh], 0.0)
    h1 = jnp.maximum(p1_ref[...] + b_ref[0, 0, dh:], 0.0)
    h = jnp.concatenate([h0, h1], axis=1)
    o_ref[0] = jnp.dot(h, w_ref[0], preferred_element_type=jnp.float32)


def _mid(p0, p1, b, w):
    _, din, dh = w.shape
    dh_in = p0.shape[1]
    return pl.pallas_call(
        _mid_body,
        grid=(_GRID, 2),
        in_specs=[
            pl.BlockSpec((_BN, dh_in), lambda i, j: (i, 0)),
            pl.BlockSpec((_BN, dh_in), lambda i, j: (i, 0)),
            pl.BlockSpec((1, 1, din), lambda i, j: (0, 0, 0)),
            pl.BlockSpec((1, din, dh), lambda i, j: (j, 0, 0)),
        ],
        out_specs=pl.BlockSpec((1, _BN, dh), lambda i, j: (j, i, 0)),
        out_shape=jax.ShapeDtypeStruct((2, N, dh), jnp.float32),
    )(p0, p1, b.reshape(1, 1, din), w)


def _out_body(p0_ref, p1_ref, b_ref, o_ref):
    logits = (jnp.concatenate([p0_ref[...], p1_ref[...]], axis=1)
              + b_ref[0, 0])
    m = jnp.max(logits, axis=1, keepdims=True)
    lse = jnp.log(jnp.sum(jnp.exp(logits - m), axis=1, keepdims=True))
    o_ref[...] = (logits - m) - lse


def _fin(p0, p1, b):
    d = 2 * p0.shape[1]
    dh = d // 2
    return pl.pallas_call(
        _out_body,
        grid=(_GRID,),
        in_specs=[
            pl.BlockSpec((_BN, dh), lambda i: (i, 0)),
            pl.BlockSpec((_BN, dh), lambda i: (i, 0)),
            pl.BlockSpec((1, 1, d), lambda i: (0, 0, 0)),
        ],
        out_specs=pl.BlockSpec((_BN, d), lambda i: (i, 0)),
        out_shape=jax.ShapeDtypeStruct((N, d), jnp.float32),
    )(p0, p1, b.reshape(1, 1, d))


def kernel(x, edge_index, edge_weight, W1, b1, W2, b2):
    row = edge_index[0]
    col = edge_index[1]

    col2 = col.reshape(NS, EPT)
    row2 = row.reshape(NS, EPT)
    w2 = edge_weight.reshape(NS, EPT)

    W1s = W1.reshape(D_IN, 2, D_H // 2).transpose(1, 0, 2)
    W2s = W2.reshape(D_H, 2, D_OUT // 2).transpose(1, 0, 2)

    xw = _mm(x, W1s)
    p1 = _spmm_h(xw, col2, row2, w2)
    hw = _mid(p1[0, :N], p1[1, :N], b1, W2s)
    p2 = _spmm_o(hw, col2, row2, w2)
    return _fin(p2[0, :N], p2[1, :N], b2)


# speedup vs baseline: 6.9459x; 1.1268x over previous
import functools

import jax
import jax.numpy as jnp
from jax import lax
from jax.experimental import pallas as pl
from jax.experimental.pallas import tpu as pltpu
from jax.experimental.pallas import tpu_sc as plsc

N = 10000
E = 320000
D_IN = 128
D_H = 128
D_OUT = 64

NC = 2
NS = 16
EPT = E // NS
K = 80
CHUNKS = EPT // K
NBUF = 5
NPAD = 10240
ROWS_PT = NPAD // NS

_DNUMS = lax.GatherDimensionNumbers(
    offset_dims=(), collapsed_slice_dims=(0,), start_index_map=(0,))


def _bcast_lane(v16, l):
    return lax.gather(v16, jnp.full((16, 1), l, jnp.int32), _DNUMS,
                      slice_sizes=(1,),
                      mode=lax.GatherScatterMode.PROMISE_IN_BOUNDS)


def _make_spmm(DH):
    mesh = plsc.VectorSubcoreMesh(core_axis_name="c", subcore_axis_name="s")

    @functools.partial(
        pl.kernel,
        out_type=jax.ShapeDtypeStruct((NC, NPAD, DH), jnp.float32),
        mesh=mesh,
        scratch_types=[
            pltpu.VMEM((EPT,), jnp.int32),
            pltpu.VMEM((EPT,), jnp.int32),
            pltpu.VMEM((EPT,), jnp.float32),
            pltpu.VMEM((NBUF, K, DH), jnp.float32),
            pltpu.VMEM_SHARED((NPAD, DH), jnp.float32),
            pltpu.SemaphoreType.DMA((NBUF,)),
            pltpu.SemaphoreType.DMA((NBUF,)),
        ] + [pltpu.VMEM((K,), jnp.int32) for _ in range(2 * NBUF)],
        compiler_params=pltpu.CompilerParams(use_tc_tiling_on_sc=False),
    )
    def spmm(xw_hbm, col_hbm, row_hbm, w_hbm, out_hbm, colv, rowv, wv, gbuf,
             acc, sem, ssem, *idxbufs):
        cidx = idxbufs[:NBUF]
        ridx = idxbufs[NBUF:]
        c = lax.axis_index("c")
        s = lax.axis_index("s")
        src = xw_hbm.at[c]

        pltpu.sync_copy(col_hbm.at[s], colv)
        pltpu.sync_copy(row_hbm.at[s], rowv)
        pltpu.sync_copy(w_hbm.at[s], wv)

        def zero_body(i, _):
            for dd in range(DH // 16):
                gbuf[0, i, pl.ds(dd * 16, 16)] = jnp.zeros((16,), jnp.float32)
            return 0

        lax.fori_loop(0, K, zero_body, 0)
        for jz in range(ROWS_PT // K):
            pltpu.sync_copy(gbuf.at[0], acc.at[pl.ds(s * ROWS_PT + jz * K, K)])
        plsc.subcore_barrier()

        def _vcopy(src_ref, j_row, dst_ref):
            for q in range(K // 16):
                dst_ref[pl.ds(q * 16, 16)] = (
                    src_ref[pl.ds(j_row * K + q * 16, 16)])

        for b in range(NBUF):
            _vcopy(colv, b, cidx[b])
            pltpu.async_copy(src.at[cidx[b]], gbuf.at[b], sem.at[b])

        def outer_body(jo, _):
            for b in range(NBUF):
                j = jo * NBUF + b
                pltpu.make_async_copy(
                    src.at[cidx[b]], gbuf.at[b], sem.at[b]).wait()

                def mul_grp(g, _):
                    w16 = wv[pl.ds(j * K + g * 16, 16)]
                    for l in range(16):
                        wb = _bcast_lane(w16, l)
                        i = g * 16 + l
                        for dd in range(DH // 16):
                            gbuf[b, i, pl.ds(dd * 16, 16)] = (
                                gbuf[b, i, pl.ds(dd * 16, 16)] * wb)
                    return 0

                lax.fori_loop(0, K // 16, mul_grp, 0)
                _vcopy(rowv, j, ridx[b])
                pltpu.async_copy(gbuf.at[b], acc.at[ridx[b]], ssem.at[b],
                                 add=True)

                bd = (b - 2) % NBUF
                jn = j + NBUF - 2

                @pl.when((j >= 2) & (jn < CHUNKS))
                def _():
                    pltpu.make_async_copy(
                        gbuf.at[bd], acc.at[ridx[bd]], ssem.at[bd]).wait()
                    _vcopy(colv, jn, cidx[bd])
                    pltpu.async_copy(src.at[cidx[bd]], gbuf.at[bd],
                                     sem.at[bd])
            return 0

        lax.fori_loop(0, CHUNKS // NBUF, outer_body, 0)

        for b in range(NBUF):
            pltpu.make_async_copy(
                gbuf.at[b], acc.at[ridx[b]], ssem.at[b]).wait()
        plsc.subcore_barrier()

        for jz in range(ROWS_PT // K):
            r0 = s * ROWS_PT + jz * K
            pltpu.sync_copy(acc.at[pl.ds(r0, K)], gbuf.at[0])
            pltpu.sync_copy(gbuf.at[0], out_hbm.at[c, pl.ds(r0, K)])

    return spmm


_spmm_h = _make_spmm(D_H // 2)
_spmm_o = _make_spmm(D_OUT // 2)


_GRID = 10
_BN = N // _GRID


def _mm_body(x_ref, w_ref, o_ref):
    o_ref[0] = jnp.dot(x_ref[...], w_ref[0],
                       preferred_element_type=jnp.float32)


def _mm(x, w):
    _, din, dh = w.shape
    return pl.pallas_call(
        _mm_body,
        grid=(_GRID, 2),
        in_specs=[
            pl.BlockSpec((_BN, din), lambda i, j: (i, 0)),
            pl.BlockSpec((1, din, dh), lambda i, j: (j, 0, 0)),
        ],
        out_specs=pl.BlockSpec((1, _BN, dh), lambda i, j: (j, i, 0)),
        out_shape=jax.ShapeDtypeStruct((2, N, dh), jnp.float32),
    )(x, w)


def _mid_body(p0_ref, p1_ref, b_ref, w_ref, o_ref):
    dh = p0_ref.shape[1]
    h0 = jnp.maximum(p0_ref[...] + b_ref[0, 0, :dh], 0.0)
    h1 = jnp.maximum(p1_ref[...] + b_ref[0, 0, dh:], 0.0)
    h = jnp.concatenate([h0, h1], axis=1)
    o_ref[0] = jnp.dot(h, w_ref[0], preferred_element_type=jnp.float32)


def _mid(p0, p1, b, w):
    _, din, dh = w.shape
    dh_in = p0.shape[1]
    return pl.pallas_call(
        _mid_body,
        grid=(_GRID, 2),
        in_specs=[
            pl.BlockSpec((_BN, dh_in), lambda i, j: (i, 0)),
            pl.BlockSpec((_BN, dh_in), lambda i, j: (i, 0)),
            pl.BlockSpec((1, 1, din), lambda i, j: (0, 0, 0)),
            pl.BlockSpec((1, din, dh), lambda i, j: (j, 0, 0)),
        ],
        out_specs=pl.BlockSpec((1, _BN, dh), lambda i, j: (j, i, 0)),
        out_shape=jax.ShapeDtypeStruct((2, N, dh), jnp.float32),
    )(p0, p1, b.reshape(1, 1, din), w)


def _out_body(p0_ref, p1_ref, b_ref, o_ref):
    logits = (jnp.concatenate([p0_ref[...], p1_ref[...]], axis=1)
              + b_ref[0, 0])
    m = jnp.max(logits, axis=1, keepdims=True)
    lse = jnp.log(jnp.sum(jnp.exp(logits - m), axis=1, keepdims=True))
    o_ref[...] = (logits - m) - lse


def _fin(p0, p1, b):
    d = 2 * p0.shape[1]
    dh = d // 2
    return pl.pallas_call(
        _out_body,
        grid=(_GRID,),
        in_specs=[
            pl.BlockSpec((_BN, dh), lambda i: (i, 0)),
            pl.BlockSpec((_BN, dh), lambda i: (i, 0)),
            pl.BlockSpec((1, 1, d), lambda i: (0, 0, 0)),
        ],
        out_specs=pl.BlockSpec((_BN, d), lambda i: (i, 0)),
        out_shape=jax.ShapeDtypeStruct((N, d), jnp.float32),
    )(p0, p1, b.reshape(1, 1, d))


def kernel(x, edge_index, edge_weight, W1, b1, W2, b2):
    row = edge_index[0]
    col = edge_index[1]

    col2 = col.reshape(NS, EPT)
    row2 = row.reshape(NS, EPT)
    w2 = edge_weight.reshape(NS, EPT)

    W1s = W1.reshape(D_IN, 2, D_H // 2).transpose(1, 0, 2)
    W2s = W2.reshape(D_H, 2, D_OUT // 2).transpose(1, 0, 2)

    xw = _mm(x, W1s)
    p1 = _spmm_h(xw, col2, row2, w2)
    hw = _mid(p1[0, :N], p1[1, :N], b1, W2s)
    p2 = _spmm_o(hw, col2, row2, w2)
    return _fin(p2[0, :N], p2[1, :N], b2)


# speedup vs baseline: 9.2870x; 1.3370x over previous
import functools

import jax
import jax.numpy as jnp
from jax import lax
from jax.experimental import pallas as pl
from jax.experimental.pallas import tpu as pltpu
from jax.experimental.pallas import tpu_sc as plsc

N = 10000
E = 320000
D_IN = 128
D_H = 128
D_OUT = 64

NC = 2
NS = 16
EPT = E // NS
K = 80
CHUNKS = EPT // K
NBUF = 5
NPAD = 10240
ROWS_PT = NPAD // NS

_DNUMS = lax.GatherDimensionNumbers(
    offset_dims=(), collapsed_slice_dims=(0,), start_index_map=(0,))


def _bcast_lane(v16, l):
    return lax.gather(v16, jnp.full((16, 1), l, jnp.int32), _DNUMS,
                      slice_sizes=(1,),
                      mode=lax.GatherScatterMode.PROMISE_IN_BOUNDS)


def _make_spmm(DH):
    mesh = plsc.VectorSubcoreMesh(core_axis_name="c", subcore_axis_name="s")

    @functools.partial(
        pl.kernel,
        out_type=jax.ShapeDtypeStruct((NC, NPAD, DH), jnp.float32),
        mesh=mesh,
        scratch_types=[
            pltpu.VMEM((EPT,), jnp.int32),
            pltpu.VMEM((EPT,), jnp.int32),
            pltpu.VMEM((EPT,), jnp.float32),
            pltpu.VMEM((NBUF, K, DH), jnp.float32),
            pltpu.VMEM_SHARED((NPAD, DH), jnp.float32),
            pltpu.SemaphoreType.DMA((NBUF,)),
            pltpu.SemaphoreType.DMA((NBUF,)),
        ] + [pltpu.VMEM((K,), jnp.int32) for _ in range(2 * NBUF)],
        compiler_params=pltpu.CompilerParams(use_tc_tiling_on_sc=False),
    )
    def spmm(xw_hbm, col_hbm, row_hbm, w_hbm, out_hbm, colv, rowv, wv, gbuf,
             acc, sem, ssem, *idxbufs):
        cidx = idxbufs[:NBUF]
        ridx = idxbufs[NBUF:]
        c = lax.axis_index("c")
        s = lax.axis_index("s")
        src = xw_hbm.at[c]

        pltpu.sync_copy(col_hbm.at[s], colv)
        pltpu.sync_copy(row_hbm.at[s], rowv)
        pltpu.sync_copy(w_hbm.at[s], wv)

        def zero_body(i, _):
            for dd in range(DH // 16):
                gbuf[0, i, pl.ds(dd * 16, 16)] = jnp.zeros((16,), jnp.float32)
            return 0

        lax.fori_loop(0, K, zero_body, 0)
        for jz in range(ROWS_PT // K):
            pltpu.sync_copy(gbuf.at[0], acc.at[pl.ds(s * ROWS_PT + jz * K, K)])
        plsc.subcore_barrier()

        def _vcopy(src_ref, j_row, dst_ref):
            for q in range(K // 16):
                dst_ref[pl.ds(q * 16, 16)] = (
                    src_ref[pl.ds(j_row * K + q * 16, 16)])

        for b in range(NBUF):
            _vcopy(colv, b, cidx[b])
            pltpu.async_copy(src.at[cidx[b]], gbuf.at[b], sem.at[b])

        def outer_body(jo, _):
            for b in range(NBUF):
                j = jo * NBUF + b
                pltpu.make_async_copy(
                    src.at[cidx[b]], gbuf.at[b], sem.at[b]).wait()

                for g in range(K // 16):
                    w16 = wv[pl.ds(j * K + g * 16, 16)]
                    for l in range(16):
                        wb = _bcast_lane(w16, l)
                        i = g * 16 + l
                        for dd in range(DH // 16):
                            gbuf[b, i, pl.ds(dd * 16, 16)] = (
                                gbuf[b, i, pl.ds(dd * 16, 16)] * wb)
                _vcopy(rowv, j, ridx[b])
                pltpu.async_copy(gbuf.at[b], acc.at[ridx[b]], ssem.at[b],
                                 add=True)

                bd = (b - 2) % NBUF
                jn = j + NBUF - 2

                @pl.when((j >= 2) & (jn < CHUNKS))
                def _():
                    pltpu.make_async_copy(
                        gbuf.at[bd], acc.at[ridx[bd]], ssem.at[bd]).wait()
                    _vcopy(colv, jn, cidx[bd])
                    pltpu.async_copy(src.at[cidx[bd]], gbuf.at[bd],
                                     sem.at[bd])
            return 0

        lax.fori_loop(0, CHUNKS // NBUF, outer_body, 0)

        for b in range(NBUF):
            pltpu.make_async_copy(
                gbuf.at[b], acc.at[ridx[b]], ssem.at[b]).wait()
        plsc.subcore_barrier()

        for jz in range(ROWS_PT // K):
            r0 = s * ROWS_PT + jz * K
            pltpu.sync_copy(acc.at[pl.ds(r0, K)], gbuf.at[0])
            pltpu.sync_copy(gbuf.at[0], out_hbm.at[c, pl.ds(r0, K)])

    return spmm


_spmm_h = _make_spmm(D_H // 2)
_spmm_o = _make_spmm(D_OUT // 2)


_GRID = 10
_BN = N // _GRID


def _mm_body(x_ref, w_ref, o_ref):
    o_ref[0] = jnp.dot(x_ref[...], w_ref[0],
                       preferred_element_type=jnp.float32)


def _mm(x, w):
    _, din, dh = w.shape
    return pl.pallas_call(
        _mm_body,
        grid=(_GRID, 2),
        in_specs=[
            pl.BlockSpec((_BN, din), lambda i, j: (i, 0)),
            pl.BlockSpec((1, din, dh), lambda i, j: (j, 0, 0)),
        ],
        out_specs=pl.BlockSpec((1, _BN, dh), lambda i, j: (j, i, 0)),
        out_shape=jax.ShapeDtypeStruct((2, N, dh), jnp.float32),
    )(x, w)


def _mid_body(p0_ref, p1_ref, b_ref, w_ref, o_ref):
    dh = p0_ref.shape[1]
    h0 = jnp.maximum(p0_ref[...] + b_ref[0, 0, :dh], 0.0)
    h1 = jnp.maximum(p1_ref[...] + b_ref[0, 0, dh:], 0.0)
    h = jnp.concatenate([h0, h1], axis=1)
    o_ref[0] = jnp.dot(h, w_ref[0], preferred_element_type=jnp.float32)


def _mid(p0, p1, b, w):
    _, din, dh = w.shape
    dh_in = p0.shape[1]
    return pl.pallas_call(
        _mid_body,
        grid=(_GRID, 2),
        in_specs=[
            pl.BlockSpec((_BN, dh_in), lambda i, j: (i, 0)),
            pl.BlockSpec((_BN, dh_in), lambda i, j: (i, 0)),
            pl.BlockSpec((1, 1, din), lambda i, j: (0, 0, 0)),
            pl.BlockSpec((1, din, dh), lambda i, j: (j, 0, 0)),
        ],
        out_specs=pl.BlockSpec((1, _BN, dh), lambda i, j: (j, i, 0)),
        out_shape=jax.ShapeDtypeStruct((2, N, dh), jnp.float32),
    )(p0, p1, b.reshape(1, 1, din), w)


def _out_body(p0_ref, p1_ref, b_ref, o_ref):
    logits = (jnp.concatenate([p0_ref[...], p1_ref[...]], axis=1)
              + b_ref[0, 0])
    m = jnp.max(logits, axis=1, keepdims=True)
    lse = jnp.log(jnp.sum(jnp.exp(logits - m), axis=1, keepdims=True))
    o_ref[...] = (logits - m) - lse


def _fin(p0, p1, b):
    d = 2 * p0.shape[1]
    dh = d // 2
    return pl.pallas_call(
        _out_body,
        grid=(_GRID,),
        in_specs=[
            pl.BlockSpec((_BN, dh), lambda i: (i, 0)),
            pl.BlockSpec((_BN, dh), lambda i: (i, 0)),
            pl.BlockSpec((1, 1, d), lambda i: (0, 0, 0)),
        ],
        out_specs=pl.BlockSpec((_BN, d), lambda i: (i, 0)),
        out_shape=jax.ShapeDtypeStruct((N, d), jnp.float32),
    )(p0, p1, b.reshape(1, 1, d))


def kernel(x, edge_index, edge_weight, W1, b1, W2, b2):
    row = edge_index[0]
    col = edge_index[1]

    col2 = col.reshape(NS, EPT)
    row2 = row.reshape(NS, EPT)
    w2 = edge_weight.reshape(NS, EPT)

    W1s = W1.reshape(D_IN, 2, D_H // 2).transpose(1, 0, 2)
    W2s = W2.reshape(D_H, 2, D_OUT // 2).transpose(1, 0, 2)

    xw = _mm(x, W1s)
    p1 = _spmm_h(xw, col2, row2, w2)
    hw = _mid(p1[0, :N], p1[1, :N], b1, W2s)
    p2 = _spmm_o(hw, col2, row2, w2)
    return _fin(p2[0, :N], p2[1, :N], b2)


# speedup vs baseline: 9.5609x; 1.0295x over previous
import functools

import jax
import jax.numpy as jnp
from jax import lax
from jax.experimental import pallas as pl
from jax.experimental.pallas import tpu as pltpu
from jax.experimental.pallas import tpu_sc as plsc

N = 10000
E = 320000
D_IN = 128
D_H = 128
D_OUT = 64

NC = 2
NS = 16
EPT = E // NS
K = 80
CHUNKS = EPT // K
NBUF = 5
NPAD = 10240
ROWS_PT = NPAD // NS

_DNUMS = lax.GatherDimensionNumbers(
    offset_dims=(), collapsed_slice_dims=(0,), start_index_map=(0,))


def _bcast_lane(v16, l):
    return lax.gather(v16, jnp.full((16, 1), l, jnp.int32), _DNUMS,
                      slice_sizes=(1,),
                      mode=lax.GatherScatterMode.PROMISE_IN_BOUNDS)


def _make_spmm(DH):
    mesh = plsc.VectorSubcoreMesh(core_axis_name="c", subcore_axis_name="s")

    @functools.partial(
        pl.kernel,
        out_type=jax.ShapeDtypeStruct((NC, NPAD, DH), jnp.float32),
        mesh=mesh,
        scratch_types=[
            pltpu.VMEM((EPT,), jnp.int32),
            pltpu.VMEM((EPT,), jnp.int32),
            pltpu.VMEM((EPT,), jnp.float32),
            pltpu.VMEM((NBUF, K, DH), jnp.float32),
            pltpu.VMEM_SHARED((NPAD, DH), jnp.float32),
            pltpu.SemaphoreType.DMA((NBUF,)),
            pltpu.SemaphoreType.DMA((NBUF,)),
        ] + [pltpu.VMEM((K,), jnp.int32) for _ in range(2 * NBUF)],
        compiler_params=pltpu.CompilerParams(use_tc_tiling_on_sc=False),
    )
    def spmm(xw_hbm, col_hbm, row_hbm, w_hbm, out_hbm, colv, rowv, wv, gbuf,
             acc, sem, ssem, *idxbufs):
        cidx = idxbufs[:NBUF]
        ridx = idxbufs[NBUF:]
        c = lax.axis_index("c")
        s = lax.axis_index("s")
        src = xw_hbm.at[c]

        pltpu.sync_copy(col_hbm.at[s], colv)
        pltpu.sync_copy(row_hbm.at[s], rowv)
        pltpu.sync_copy(w_hbm.at[s], wv)

        def zero_body(i, _):
            for dd in range(DH // 16):
                gbuf[0, i, pl.ds(dd * 16, 16)] = jnp.zeros((16,), jnp.float32)
            return 0

        lax.fori_loop(0, K, zero_body, 0)
        for jz in range(ROWS_PT // K):
            pltpu.sync_copy(gbuf.at[0], acc.at[pl.ds(s * ROWS_PT + jz * K, K)])
        plsc.subcore_barrier()

        def _vcopy(src_ref, j_row, dst_ref):
            for q in range(K // 16):
                dst_ref[pl.ds(q * 16, 16)] = (
                    src_ref[pl.ds(j_row * K + q * 16, 16)])

        for b in range(NBUF):
            _vcopy(colv, b, cidx[b])
            pltpu.async_copy(src.at[cidx[b]], gbuf.at[b], sem.at[b])

        def outer_body(jo, _):
            for b in range(NBUF):
                j = jo * NBUF + b
                pltpu.make_async_copy(
                    src.at[cidx[b]], gbuf.at[b], sem.at[b]).wait()

                for g in range(K // 16):
                    w16 = wv[pl.ds(j * K + g * 16, 16)]
                    for l in range(16):
                        wb = _bcast_lane(w16, l)
                        i = g * 16 + l
                        for dd in range(DH // 16):
                            gbuf[b, i, pl.ds(dd * 16, 16)] = (
                                gbuf[b, i, pl.ds(dd * 16, 16)] * wb)
                _vcopy(rowv, j, ridx[b])
                pltpu.async_copy(gbuf.at[b], acc.at[ridx[b]], ssem.at[b],
                                 add=True)

                bd = (b - 2) % NBUF
                jn = j + NBUF - 2

                @pl.when((j >= 2) & (jn < CHUNKS))
                def _():
                    pltpu.make_async_copy(
                        gbuf.at[bd], acc.at[ridx[bd]], ssem.at[bd]).wait()
                    _vcopy(colv, jn, cidx[bd])
                    pltpu.async_copy(src.at[cidx[bd]], gbuf.at[bd],
                                     sem.at[bd])
            return 0

        lax.fori_loop(0, CHUNKS // NBUF, outer_body, 0)

        for b in range(NBUF):
            pltpu.make_async_copy(
                gbuf.at[b], acc.at[ridx[b]], ssem.at[b]).wait()
        plsc.subcore_barrier()

        for jz in range(ROWS_PT // K):
            r0 = s * ROWS_PT + jz * K
            pltpu.sync_copy(acc.at[pl.ds(r0, K)], gbuf.at[0])
            pltpu.sync_copy(gbuf.at[0], out_hbm.at[c, pl.ds(r0, K)])

    return spmm


_spmm_h = _make_spmm(D_H // 2)
_spmm_o = _make_spmm(D_OUT // 2)


_GRID = 10
_BN = N // _GRID


def _mm_body(x_ref, w_ref, o_ref):
    o_ref[0] = jnp.dot(x_ref[...], w_ref[0],
                       preferred_element_type=jnp.float32)


def _mm(x, w):
    _, din, dh = w.shape
    return pl.pallas_call(
        _mm_body,
        grid=(_GRID, 2),
        in_specs=[
            pl.BlockSpec((_BN, din), lambda i, j: (i, 0)),
            pl.BlockSpec((1, din, dh), lambda i, j: (j, 0, 0)),
        ],
        out_specs=pl.BlockSpec((1, _BN, dh), lambda i, j: (j, i, 0)),
        out_shape=jax.ShapeDtypeStruct((2, N, dh), jnp.float32),
    )(x, w)


def _mid_body(p0_ref, p1_ref, b_ref, w_ref, o_ref):
    dh = p0_ref.shape[2]
    h0 = jnp.maximum(p0_ref[0] + b_ref[0, 0, :dh], 0.0)
    h1 = jnp.maximum(p1_ref[0] + b_ref[0, 0, dh:], 0.0)
    h = jnp.concatenate([h0, h1], axis=1)
    o_ref[0] = jnp.dot(h, w_ref[0], preferred_element_type=jnp.float32)


def _mid(p0, b, w):
    _, din, dh = w.shape
    dh_in = p0.shape[2]
    return pl.pallas_call(
        _mid_body,
        grid=(_GRID, 2),
        in_specs=[
            pl.BlockSpec((1, _BN, dh_in), lambda i, j: (0, i, 0)),
            pl.BlockSpec((1, _BN, dh_in), lambda i, j: (1, i, 0)),
            pl.BlockSpec((1, 1, din), lambda i, j: (0, 0, 0)),
            pl.BlockSpec((1, din, dh), lambda i, j: (j, 0, 0)),
        ],
        out_specs=pl.BlockSpec((1, _BN, dh), lambda i, j: (j, i, 0)),
        out_shape=jax.ShapeDtypeStruct((2, N, dh), jnp.float32),
    )(p0, p0, b.reshape(1, 1, din), w)


def _out_body(p0_ref, p1_ref, b_ref, o_ref):
    logits = (jnp.concatenate([p0_ref[0], p1_ref[0]], axis=1)
              + b_ref[0, 0])
    m = jnp.max(logits, axis=1, keepdims=True)
    lse = jnp.log(jnp.sum(jnp.exp(logits - m), axis=1, keepdims=True))
    o_ref[...] = (logits - m) - lse


def _fin(p0, b):
    dh = p0.shape[2]
    d = 2 * dh
    return pl.pallas_call(
        _out_body,
        grid=(_GRID,),
        in_specs=[
            pl.BlockSpec((1, _BN, dh), lambda i: (0, i, 0)),
            pl.BlockSpec((1, _BN, dh), lambda i: (1, i, 0)),
            pl.BlockSpec((1, 1, d), lambda i: (0, 0, 0)),
        ],
        out_specs=pl.BlockSpec((_BN, d), lambda i: (i, 0)),
        out_shape=jax.ShapeDtypeStruct((N, d), jnp.float32),
    )(p0, p0, b.reshape(1, 1, d))


def kernel(x, edge_index, edge_weight, W1, b1, W2, b2):
    row = edge_index[0]
    col = edge_index[1]

    col2 = col.reshape(NS, EPT)
    row2 = row.reshape(NS, EPT)
    w2 = edge_weight.reshape(NS, EPT)

    W1s = W1.reshape(D_IN, 2, D_H // 2).transpose(1, 0, 2)
    W2s = W2.reshape(D_H, 2, D_OUT // 2).transpose(1, 0, 2)

    xw = _mm(x, W1s)
    p1 = _spmm_h(xw, col2, row2, w2)
    hw = _mid(p1, b1, W2s)
    p2 = _spmm_o(hw, col2, row2, w2)
    return _fin(p2, b2)


# speedup vs baseline: 9.5741x; 1.0014x over previous
import functools

import jax
import jax.numpy as jnp
from jax import lax
from jax.experimental import pallas as pl
from jax.experimental.pallas import tpu as pltpu
from jax.experimental.pallas import tpu_sc as plsc

N = 10000
E = 320000
D_IN = 128
D_H = 128
D_OUT = 64

NC = 2
NS = 16
EPT = E // NS
K = 80
CHUNKS = EPT // K
NBUF = 5
NPAD = 10240
ROWS_PT = NPAD // NS

_DNUMS = lax.GatherDimensionNumbers(
    offset_dims=(), collapsed_slice_dims=(0,), start_index_map=(0,))


def _bcast_lane(v16, l):
    return lax.gather(v16, jnp.full((16, 1), l, jnp.int32), _DNUMS,
                      slice_sizes=(1,),
                      mode=lax.GatherScatterMode.PROMISE_IN_BOUNDS)


def _make_spmm(DH):
    mesh = plsc.VectorSubcoreMesh(core_axis_name="c", subcore_axis_name="s")

    @functools.partial(
        pl.kernel,
        out_type=jax.ShapeDtypeStruct((NC, NPAD, DH), jnp.float32),
        mesh=mesh,
        scratch_types=[
            pltpu.VMEM((EPT,), jnp.int32),
            pltpu.VMEM((EPT,), jnp.int32),
            pltpu.VMEM((EPT,), jnp.float32),
            pltpu.VMEM((NBUF, K, DH), jnp.float32),
            pltpu.VMEM_SHARED((NPAD, DH), jnp.float32),
            pltpu.SemaphoreType.DMA((NBUF,)),
            pltpu.SemaphoreType.DMA((NBUF,)),
        ] + [pltpu.VMEM((K,), jnp.int32) for _ in range(2 * NBUF)],
        compiler_params=pltpu.CompilerParams(use_tc_tiling_on_sc=False, disable_bounds_checks=True),
    )
    def spmm(xw_hbm, col_hbm, row_hbm, w_hbm, out_hbm, colv, rowv, wv, gbuf,
             acc, sem, ssem, *idxbufs):
        cidx = idxbufs[:NBUF]
        ridx = idxbufs[NBUF:]
        c = lax.axis_index("c")
        s = lax.axis_index("s")
        src = xw_hbm.at[c]

        pltpu.sync_copy(col_hbm.at[s], colv)
        pltpu.sync_copy(row_hbm.at[s], rowv)
        pltpu.sync_copy(w_hbm.at[s], wv)

        def zero_body(i, _):
            for dd in range(DH // 16):
                gbuf[0, i, pl.ds(dd * 16, 16)] = jnp.zeros((16,), jnp.float32)
            return 0

        lax.fori_loop(0, K, zero_body, 0)
        for jz in range(ROWS_PT // K):
            pltpu.sync_copy(gbuf.at[0], acc.at[pl.ds(s * ROWS_PT + jz * K, K)])
        plsc.subcore_barrier()

        def _vcopy(src_ref, j_row, dst_ref):
            for q in range(K // 16):
                dst_ref[pl.ds(q * 16, 16)] = (
                    src_ref[pl.ds(j_row * K + q * 16, 16)])

        for b in range(NBUF):
            _vcopy(colv, b, cidx[b])
            pltpu.async_copy(src.at[cidx[b]], gbuf.at[b], sem.at[b])

        def outer_body(jo, _):
            for b in range(NBUF):
                j = jo * NBUF + b
                pltpu.make_async_copy(
                    src.at[cidx[b]], gbuf.at[b], sem.at[b]).wait()

                for g in range(K // 16):
                    w16 = wv[pl.ds(j * K + g * 16, 16)]
                    for l in range(16):
                        wb = _bcast_lane(w16, l)
                        i = g * 16 + l
                        for dd in range(DH // 16):
                            gbuf[b, i, pl.ds(dd * 16, 16)] = (
                                gbuf[b, i, pl.ds(dd * 16, 16)] * wb)
                _vcopy(rowv, j, ridx[b])
                pltpu.async_copy(gbuf.at[b], acc.at[ridx[b]], ssem.at[b],
                                 add=True)

                bd = (b - 2) % NBUF
                jn = j + NBUF - 2

                @pl.when((j >= 2) & (jn < CHUNKS))
                def _():
                    pltpu.make_async_copy(
                        gbuf.at[bd], acc.at[ridx[bd]], ssem.at[bd]).wait()
                    _vcopy(colv, jn, cidx[bd])
                    pltpu.async_copy(src.at[cidx[bd]], gbuf.at[bd],
                                     sem.at[bd])
            return 0

        lax.fori_loop(0, CHUNKS // NBUF, outer_body, 0)

        for b in range(NBUF):
            pltpu.make_async_copy(
                gbuf.at[b], acc.at[ridx[b]], ssem.at[b]).wait()
        plsc.subcore_barrier()

        for jz in range(ROWS_PT // K):
            r0 = s * ROWS_PT + jz * K
            pltpu.sync_copy(acc.at[pl.ds(r0, K)], gbuf.at[0])
            pltpu.sync_copy(gbuf.at[0], out_hbm.at[c, pl.ds(r0, K)])

    return spmm


_spmm_h = _make_spmm(D_H // 2)
_spmm_o = _make_spmm(D_OUT // 2)


_GRID = 10
_BN = N // _GRID


def _mm_body(x_ref, w_ref, o_ref):
    o_ref[0] = jnp.dot(x_ref[...], w_ref[0],
                       preferred_element_type=jnp.float32)


def _mm(x, w):
    _, din, dh = w.shape
    return pl.pallas_call(
        _mm_body,
        grid=(_GRID, 2),
        in_specs=[
            pl.BlockSpec((_BN, din), lambda i, j: (i, 0)),
            pl.BlockSpec((1, din, dh), lambda i, j: (j, 0, 0)),
        ],
        out_specs=pl.BlockSpec((1, _BN, dh), lambda i, j: (j, i, 0)),
        out_shape=jax.ShapeDtypeStruct((2, N, dh), jnp.float32),
    )(x, w)


def _mid_body(p0_ref, p1_ref, b_ref, w_ref, o_ref):
    dh = p0_ref.shape[2]
    h0 = jnp.maximum(p0_ref[0] + b_ref[0, 0, :dh], 0.0)
    h1 = jnp.maximum(p1_ref[0] + b_ref[0, 0, dh:], 0.0)
    h = jnp.concatenate([h0, h1], axis=1)
    o_ref[0] = jnp.dot(h, w_ref[0], preferred_element_type=jnp.float32)


def _mid(p0, b, w):
    _, din, dh = w.shape
    dh_in = p0.shape[2]
    return pl.pallas_call(
        _mid_body,
        grid=(_GRID, 2),
        in_specs=[
            pl.BlockSpec((1, _BN, dh_in), lambda i, j: (0, i, 0)),
            pl.BlockSpec((1, _BN, dh_in), lambda i, j: (1, i, 0)),
            pl.BlockSpec((1, 1, din), lambda i, j: (0, 0, 0)),
            pl.BlockSpec((1, din, dh), lambda i, j: (j, 0, 0)),
        ],
        out_specs=pl.BlockSpec((1, _BN, dh), lambda i, j: (j, i, 0)),
        out_shape=jax.ShapeDtypeStruct((2, N, dh), jnp.float32),
    )(p0, p0, b.reshape(1, 1, din), w)


def _out_body(p0_ref, p1_ref, b_ref, o_ref):
    logits = (jnp.concatenate([p0_ref[0], p1_ref[0]], axis=1)
              + b_ref[0, 0])
    m = jnp.max(logits, axis=1, keepdims=True)
    lse = jnp.log(jnp.sum(jnp.exp(logits - m), axis=1, keepdims=True))
    o_ref[...] = (logits - m) - lse


def _fin(p0, b):
    dh = p0.shape[2]
    d = 2 * dh
    return pl.pallas_call(
        _out_body,
        grid=(_GRID,),
        in_specs=[
            pl.BlockSpec((1, _BN, dh), lambda i: (0, i, 0)),
            pl.BlockSpec((1, _BN, dh), lambda i: (1, i, 0)),
            pl.BlockSpec((1, 1, d), lambda i: (0, 0, 0)),
        ],
        out_specs=pl.BlockSpec((_BN, d), lambda i: (i, 0)),
        out_shape=jax.ShapeDtypeStruct((N, d), jnp.float32),
    )(p0, p0, b.reshape(1, 1, d))


def kernel(x, edge_index, edge_weight, W1, b1, W2, b2):
    row = edge_index[0]
    col = edge_index[1]

    col2 = col.reshape(NS, EPT)
    row2 = row.reshape(NS, EPT)
    w2 = edge_weight.reshape(NS, EPT)

    W1s = W1.reshape(D_IN, 2, D_H // 2).transpose(1, 0, 2)
    W2s = W2.reshape(D_H, 2, D_OUT // 2).transpose(1, 0, 2)

    xw = _mm(x, W1s)
    p1 = _spmm_h(xw, col2, row2, w2)
    hw = _mid(p1, b1, W2s)
    p2 = _spmm_o(hw, col2, row2, w2)
    return _fin(p2, b2)


# speedup vs baseline: 9.8128x; 1.0249x over previous
import functools

import jax
import jax.numpy as jnp
from jax import lax
from jax.experimental import pallas as pl
from jax.experimental.pallas import tpu as pltpu
from jax.experimental.pallas import tpu_sc as plsc

N = 10000
E = 320000
D_IN = 128
D_H = 128
D_OUT = 64

NC = 2
NS = 16
EPT = E // NS
K = 80
CHUNKS = EPT // K
NBUF = 5
NPAD = 10240
ROWS_PT = NPAD // NS

_DNUMS = lax.GatherDimensionNumbers(
    offset_dims=(), collapsed_slice_dims=(0,), start_index_map=(0,))


def _bcast_lane(v16, l):
    return lax.gather(v16, jnp.full((16, 1), l, jnp.int32), _DNUMS,
                      slice_sizes=(1,),
                      mode=lax.GatherScatterMode.PROMISE_IN_BOUNDS)


def _make_spmm(DH):
    mesh = plsc.VectorSubcoreMesh(core_axis_name="c", subcore_axis_name="s")

    @functools.partial(
        pl.kernel,
        out_type=jax.ShapeDtypeStruct((NC, NPAD, DH), jnp.float32),
        mesh=mesh,
        scratch_types=[
            pltpu.VMEM((EPT,), jnp.int32),
            pltpu.VMEM((EPT,), jnp.int32),
            pltpu.VMEM((EPT,), jnp.float32),
            pltpu.VMEM((NBUF, K, DH), jnp.float32),
            pltpu.VMEM_SHARED((NPAD, DH), jnp.float32),
            pltpu.SemaphoreType.DMA((NBUF,)),
            pltpu.SemaphoreType.DMA((NBUF,)),
        ] + [pltpu.VMEM((K,), jnp.int32) for _ in range(2 * NBUF)],
        compiler_params=pltpu.CompilerParams(use_tc_tiling_on_sc=False, disable_bounds_checks=True),
    )
    def spmm(xw_hbm, col_hbm, row_hbm, w_hbm, out_hbm, colv, rowv, wv, gbuf,
             acc, sem, ssem, *idxbufs):
        cidx = idxbufs[:NBUF]
        ridx = idxbufs[NBUF:]
        c = lax.axis_index("c")
        s = lax.axis_index("s")
        src = xw_hbm.at[c]

        pltpu.async_copy(col_hbm.at[s], colv, sem.at[0])
        pltpu.async_copy(row_hbm.at[s], rowv, sem.at[1])
        pltpu.async_copy(w_hbm.at[s], wv, sem.at[2])

        def zero_body(i, _):
            for dd in range(DH // 16):
                gbuf[0, i, pl.ds(dd * 16, 16)] = jnp.zeros((16,), jnp.float32)
            return 0

        lax.fori_loop(0, K, zero_body, 0)
        for jz in range(ROWS_PT // K):
            pltpu.sync_copy(gbuf.at[0], acc.at[pl.ds(s * ROWS_PT + jz * K, K)])
        pltpu.make_async_copy(col_hbm.at[s], colv, sem.at[0]).wait()
        pltpu.make_async_copy(row_hbm.at[s], rowv, sem.at[1]).wait()
        pltpu.make_async_copy(w_hbm.at[s], wv, sem.at[2]).wait()
        plsc.subcore_barrier()

        def _vcopy(src_ref, j_row, dst_ref):
            for q in range(K // 16):
                dst_ref[pl.ds(q * 16, 16)] = (
                    src_ref[pl.ds(j_row * K + q * 16, 16)])

        for b in range(NBUF):
            _vcopy(colv, b, cidx[b])
            pltpu.async_copy(src.at[cidx[b]], gbuf.at[b], sem.at[b])

        def outer_body(jo, _):
            for b in range(NBUF):
                j = jo * NBUF + b
                pltpu.make_async_copy(
                    src.at[cidx[b]], gbuf.at[b], sem.at[b]).wait()

                for g in range(K // 16):
                    w16 = wv[pl.ds(j * K + g * 16, 16)]
                    for l in range(16):
                        wb = _bcast_lane(w16, l)
                        i = g * 16 + l
                        for dd in range(DH // 16):
                            gbuf[b, i, pl.ds(dd * 16, 16)] = (
                                gbuf[b, i, pl.ds(dd * 16, 16)] * wb)
                _vcopy(rowv, j, ridx[b])
                pltpu.async_copy(gbuf.at[b], acc.at[ridx[b]], ssem.at[b],
                                 add=True)

                bd = (b - 2) % NBUF
                jn = j + NBUF - 2

                @pl.when((j >= 2) & (jn < CHUNKS))
                def _():
                    pltpu.make_async_copy(
                        gbuf.at[bd], acc.at[ridx[bd]], ssem.at[bd]).wait()
                    _vcopy(colv, jn, cidx[bd])
                    pltpu.async_copy(src.at[cidx[bd]], gbuf.at[bd],
                                     sem.at[bd])
            return 0

        lax.fori_loop(0, CHUNKS // NBUF, outer_body, 0)

        for b in range(NBUF):
            pltpu.make_async_copy(
                gbuf.at[b], acc.at[ridx[b]], ssem.at[b]).wait()
        plsc.subcore_barrier()

        for jz in range(ROWS_PT // K):
            bz = jz % 2
            r0 = s * ROWS_PT + jz * K
            if jz >= 2:
                pltpu.make_async_copy(
                    gbuf.at[bz], out_hbm.at[c, pl.ds(r0 - 2 * K, K)],
                    ssem.at[bz]).wait()
            pltpu.sync_copy(acc.at[pl.ds(r0, K)], gbuf.at[bz])
            pltpu.async_copy(gbuf.at[bz], out_hbm.at[c, pl.ds(r0, K)],
                             ssem.at[bz])
        for jz in (ROWS_PT // K - 2, ROWS_PT // K - 1):
            bz = jz % 2
            r0 = s * ROWS_PT + jz * K
            pltpu.make_async_copy(
                gbuf.at[bz], out_hbm.at[c, pl.ds(r0, K)], ssem.at[bz]).wait()

    return spmm


_spmm_h = _make_spmm(D_H // 2)
_spmm_o = _make_spmm(D_OUT // 2)


_GRID = 10
_BN = N // _GRID


def _mm_body(x_ref, w_ref, o_ref):
    o_ref[0] = jnp.dot(x_ref[...], w_ref[0],
                       preferred_element_type=jnp.float32)


def _mm(x, w):
    _, din, dh = w.shape
    return pl.pallas_call(
        _mm_body,
        grid=(_GRID, 2),
        in_specs=[
            pl.BlockSpec((_BN, din), lambda i, j: (i, 0)),
            pl.BlockSpec((1, din, dh), lambda i, j: (j, 0, 0)),
        ],
        out_specs=pl.BlockSpec((1, _BN, dh), lambda i, j: (j, i, 0)),
        out_shape=jax.ShapeDtypeStruct((2, N, dh), jnp.float32),
    )(x, w)


def _mid_body(p0_ref, p1_ref, b_ref, w_ref, o_ref):
    dh = p0_ref.shape[2]
    h0 = jnp.maximum(p0_ref[0] + b_ref[0, 0, :dh], 0.0)
    h1 = jnp.maximum(p1_ref[0] + b_ref[0, 0, dh:], 0.0)
    h = jnp.concatenate([h0, h1], axis=1)
    o_ref[0] = jnp.dot(h, w_ref[0], preferred_element_type=jnp.float32)


def _mid(p0, b, w):
    _, din, dh = w.shape
    dh_in = p0.shape[2]
    return pl.pallas_call(
        _mid_body,
        grid=(_GRID, 2),
        in_specs=[
            pl.BlockSpec((1, _BN, dh_in), lambda i, j: (0, i, 0)),
            pl.BlockSpec((1, _BN, dh_in), lambda i, j: (1, i, 0)),
            pl.BlockSpec((1, 1, din), lambda i, j: (0, 0, 0)),
            pl.BlockSpec((1, din, dh), lambda i, j: (j, 0, 0)),
        ],
        out_specs=pl.BlockSpec((1, _BN, dh), lambda i, j: (j, i, 0)),
        out_shape=jax.ShapeDtypeStruct((2, N, dh), jnp.float32),
    )(p0, p0, b.reshape(1, 1, din), w)


def _out_body(p0_ref, p1_ref, b_ref, o_ref):
    logits = (jnp.concatenate([p0_ref[0], p1_ref[0]], axis=1)
              + b_ref[0, 0])
    m = jnp.max(logits, axis=1, keepdims=True)
    lse = jnp.log(jnp.sum(jnp.exp(logits - m), axis=1, keepdims=True))
    o_ref[...] = (logits - m) - lse


def _fin(p0, b):
    dh = p0.shape[2]
    d = 2 * dh
    return pl.pallas_call(
        _out_body,
        grid=(_GRID,),
        in_specs=[
            pl.BlockSpec((1, _BN, dh), lambda i: (0, i, 0)),
            pl.BlockSpec((1, _BN, dh), lambda i: (1, i, 0)),
            pl.BlockSpec((1, 1, d), lambda i: (0, 0, 0)),
        ],
        out_specs=pl.BlockSpec((_BN, d), lambda i: (i, 0)),
        out_shape=jax.ShapeDtypeStruct((N, d), jnp.float32),
    )(p0, p0, b.reshape(1, 1, d))


def kernel(x, edge_index, edge_weight, W1, b1, W2, b2):
    row = edge_index[0]
    col = edge_index[1]

    col2 = col.reshape(NS, EPT)
    row2 = row.reshape(NS, EPT)
    w2 = edge_weight.reshape(NS, EPT)

    W1s = W1.reshape(D_IN, 2, D_H // 2).transpose(1, 0, 2)
    W2s = W2.reshape(D_H, 2, D_OUT // 2).transpose(1, 0, 2)

    xw = _mm(x, W1s)
    p1 = _spmm_h(xw, col2, row2, w2)
    hw = _mid(p1, b1, W2s)
    p2 = _spmm_o(hw, col2, row2, w2)
    return _fin(p2, b2)
